# baseline (speedup 1.0000x reference)
import jax
import jax.numpy as jnp
from jax.experimental import pallas as pl


def _layer(h, src, dst, W, al, ar, b):
    n = h.shape[0]
    nh, dh = al.shape
    feat = (h @ W).reshape(n, nh, dh)
    el = jnp.sum(feat * al[None, :, :], axis=-1)
    er = jnp.sum(feat * ar[None, :, :], axis=-1)
    e = el[src] + er[dst]
    e = jnp.where(e > 0, e, 0.2 * e)
    emax = jax.ops.segment_max(e, dst, num_segments=n)
    emax = jnp.where(jnp.isfinite(emax), emax, 0.0)
    ee = jnp.exp(e - emax[dst])
    denom = jax.ops.segment_sum(ee, dst, num_segments=n)
    alpha = ee / denom[dst]
    msg = feat[src] * alpha[:, :, None]
    out = jax.ops.segment_sum(msg, dst, num_segments=n)
    return out + b.reshape(1, nh, dh)


def _final_mm(hg, Wc, bc):
    def body(hg_ref, wc_ref, bc_ref, o_ref):
        o_ref[...] = jnp.dot(hg_ref[...], wc_ref[...],
                             preferred_element_type=jnp.float32) + bc_ref[...]
    return pl.pallas_call(
        body,
        out_shape=jax.ShapeDtypeStruct((1, Wc.shape[1]), jnp.float32),
    )(hg, Wc, bc.reshape(1, -1))


def kernel(x, edge_index, W1, al1, ar1, b1, W2, al2, ar2, b2, Wc, bc):
    src = edge_index[0]
    dst = edge_index[1]
    h = _layer(x, src, dst, W1, al1, ar1, b1)
    h = jax.nn.relu(h).reshape(x.shape[0], -1)
    h = _layer(h, src, dst, W2, al2, ar2, b2)
    h = h.reshape(x.shape[0], -1)
    hg = jnp.mean(h, axis=0, keepdims=True)
    return _final_mm(hg, Wc, bc)


# speedup vs baseline: 30.2331x; 30.2331x over previous
import functools

import jax
import jax.numpy as jnp
from jax import lax
from jax.experimental import pallas as pl
from jax.experimental.pallas import tpu as pltpu
from jax.experimental.pallas import tpu_sc as plsc

_H = 8
_DH = 16
_HD = _H * _DH
_NCORE = 2
_NSUB = 16
_CH = 80
_BIG = 1e30


def _prep_body(h_ref, w256_ref, bigpad_ref, fe_ref, aux_ref, c_ref):
    big = jnp.dot(h_ref[...], w256_ref[...], preferred_element_type=jnp.float32)
    fe_ref[...] = big
    aux = big[:, _HD:2 * _HD]
    aux_ref[...] = aux
    cmax = jnp.max(aux, axis=0, keepdims=True)
    csh = jnp.dot(cmax, jnp.eye(_HD, k=-_H, dtype=jnp.float32),
                  preferred_element_type=jnp.float32)
    c_ref[...] = jnp.maximum(cmax + csh, 0.0) + bigpad_ref[...]


def _prep2_body(a_ref, b_ref, w256_ref, bigpad_ref, fe_ref, aux_ref, c_ref):
    n = fe_ref.shape[0]
    h = jnp.maximum(a_ref[0, :n] + a_ref[1, :n] + b_ref[...], 0.0)
    big = jnp.dot(h, w256_ref[...], preferred_element_type=jnp.float32)
    fe_ref[...] = big
    aux = big[:, _HD:2 * _HD]
    aux_ref[...] = aux
    cmax = jnp.max(aux, axis=0, keepdims=True)
    csh = jnp.dot(cmax, jnp.eye(_HD, k=-_H, dtype=jnp.float32),
                  preferred_element_type=jnp.float32)
    c_ref[...] = jnp.maximum(cmax + csh, 0.0) + bigpad_ref[...]


def _rbuild_body(dp_ref, aux_ref, g_ref):
    n = aux_ref.shape[0]
    d = dp_ref[0, :n] + dp_ref[1, :n]
    f32 = jnp.float32
    er_sh = jnp.dot(aux_ref[...], jnp.eye(_HD, k=-_H, dtype=f32),
                    preferred_element_type=f32)
    dsh = jnp.dot(d, jnp.eye(_HD, k=_H, dtype=f32),
                  preferred_element_type=f32)
    lane = lax.broadcasted_iota(jnp.int32, d.shape, 1)
    g_ref[...] = er_sh + jnp.where((lane >= _H) & (lane < 2 * _H),
                                   1.0 / dsh, 0.0)


def _final_body(n, ap_ref, b_ref, wc_ref, bc_ref, o_ref):
    hmean = (jnp.sum(ap_ref[0] + ap_ref[1], axis=0, keepdims=True)
             * (1.0 / n) + b_ref[...])
    o_ref[...] = jnp.dot(hmean, wc_ref[...],
                         preferred_element_type=jnp.float32) + bc_ref[...]


def _tc_call(body, out_shapes, *args):
    return pl.pallas_call(body, out_shape=out_shapes)(*args)


def _vgather(vec, idx):
    dn = lax.GatherDimensionNumbers(offset_dims=(), collapsed_slice_dims=(0,),
                                    start_index_map=(0,))
    return lax.gather(vec, idx.reshape(16, 1), dn, (1,),
                      mode=lax.GatherScatterMode.PROMISE_IN_BOUNDS)


def _idx8():
    return (lax.iota(jnp.int32, 16) & 7) + _H


def _zero_rows(buf, nrows, ncols):
    def zb(j, carry):
        for col in range(0, ncols, 16):
            buf[j, pl.ds(col, 16)] = jnp.zeros((16,), jnp.float32)
        return carry
    lax.fori_loop(0, nrows, zb, 0)


def _sc_pass1_body(esrc, edst, aux_tab, c_tab, dout,
                   src_i, dst_i, s_rows, d_rows, w_rows, c_buf, dsh,
                   sem1, sem2):
    e = esrc.shape[0]
    npad = dsh.shape[0]
    ept = e // (_NCORE * _NSUB)
    nchunk = ept // _CH
    rpt = npad // _NSUB
    ci = lax.axis_index("c")
    si = lax.axis_index("s")

    _zero_rows(w_rows, _CH, _HD)
    for k in range(rpt // _CH):
        pltpu.sync_copy(w_rows, dsh.at[pl.ds(si * rpt + k * _CH, _CH)])
    plsc.subcore_barrier()

    pltpu.sync_copy(c_tab.at[0], c_buf)
    cvec = c_buf[pl.ds(0, 16)]
    idx8 = _idx8()

    def chunk(i, carry):
        base = ci * (e // _NCORE) + si * ept + i * _CH
        pltpu.sync_copy(esrc.at[pl.ds(base, _CH)], src_i)
        pltpu.sync_copy(edst.at[pl.ds(base, _CH)], dst_i)
        cp1 = pltpu.async_copy(aux_tab.at[src_i], s_rows, sem1)
        cp2 = pltpu.async_copy(aux_tab.at[dst_i], d_rows, sem2)
        cp1.wait()
        cp2.wait()

        def edge(j, c2):
            s16 = s_rows[j, pl.ds(0, 16)]
            d16 = d_rows[j, pl.ds(0, 16)]
            z = s16 + _vgather(d16, idx8)
            z = jnp.where(z > 0, z, 0.2 * z)
            w_rows[j, pl.ds(0, 16)] = jnp.exp(z - cvec)
            return c2
        lax.fori_loop(0, _CH, edge, 0)
        pltpu.sync_copy(w_rows, dsh.at[dst_i], add=True)
        return carry
    lax.fori_loop(0, nchunk, chunk, 0)
    plsc.subcore_barrier()

    for k in range(rpt // _CH):
        pltpu.sync_copy(dsh.at[pl.ds(si * rpt + k * _CH, _CH)], s_rows)
        pltpu.sync_copy(s_rows, dout.at[ci, pl.ds(si * rpt + k * _CH, _CH)])


def _sc_pass2_body(esrc, edst, fe_tab, g_tab, c_tab, aout,
                   src_i, dst_i, fe_rows, g_rows, m_rows, c_buf,
                   ash, sem1, sem2):
    e = esrc.shape[0]
    npad = ash.shape[0]
    ept = e // (_NCORE * _NSUB)
    nchunk = ept // _CH
    rpt = npad // _NSUB
    ci = lax.axis_index("c")
    si = lax.axis_index("s")

    _zero_rows(m_rows, _CH, _HD)
    for k in range(rpt // _CH):
        pltpu.sync_copy(m_rows, ash.at[pl.ds(si * rpt + k * _CH, _CH)])
    plsc.subcore_barrier()

    pltpu.sync_copy(c_tab.at[0], c_buf)
    cvec = c_buf[pl.ds(0, 16)]
    idx8 = _idx8()

    def chunk(i, carry):
        base = ci * (e // _NCORE) + si * ept + i * _CH
        pltpu.sync_copy(esrc.at[pl.ds(base, _CH)], src_i)
        pltpu.sync_copy(edst.at[pl.ds(base, _CH)], dst_i)
        cp1 = pltpu.async_copy(fe_tab.at[src_i], fe_rows, sem1)
        cp2 = pltpu.async_copy(g_tab.at[dst_i], g_rows, sem2)
        cp1.wait()
        cp2.wait()

        def edge(j, c2):
            s16 = fe_rows[j, pl.ds(_HD, 16)]
            g16 = g_rows[j, pl.ds(0, 16)]
            z = s16 + g16
            z = jnp.where(z > 0, z, 0.2 * z)
            w = jnp.exp(z - cvec)
            rd = _vgather(g16, idx8)
            coef = w * rd
            for hh in range(_H):
                b = _vgather(coef, jnp.full((16,), hh, jnp.int32))
                m_rows[j, pl.ds(hh * _DH, _DH)] = (
                    fe_rows[j, pl.ds(hh * _DH, _DH)] * b)
            return c2
        lax.fori_loop(0, _CH, edge, 0)
        pltpu.sync_copy(m_rows, ash.at[dst_i], add=True)
        return carry
    lax.fori_loop(0, nchunk, chunk, 0)
    plsc.subcore_barrier()

    for k in range(rpt // _CH):
        pltpu.sync_copy(ash.at[pl.ds(si * rpt + k * _CH, _CH)], m_rows)
        pltpu.sync_copy(m_rows, aout.at[ci, pl.ds(si * rpt + k * _CH, _CH)])


def _sc_mesh():
    return plsc.VectorSubcoreMesh(core_axis_name="c", subcore_axis_name="s",
                                  num_cores=_NCORE, num_subcores=_NSUB)


def _padn(n):
    g = _NSUB * _CH
    return ((n + g - 1) // g) * g


def _sc_pass1(esrc, edst, aux_tab, c_tab):
    npad = _padn(aux_tab.shape[0])
    rpt = npad // _NSUB
    f = pl.kernel(
        _sc_pass1_body,
        out_type=jax.ShapeDtypeStruct((_NCORE, npad, _HD), jnp.float32),
        mesh=_sc_mesh(),
        scratch_types=[
            pltpu.VMEM((_CH,), jnp.int32),
            pltpu.VMEM((_CH,), jnp.int32),
            pltpu.VMEM((_CH, _HD), jnp.float32),
            pltpu.VMEM((_CH, _HD), jnp.float32),
            pltpu.VMEM((_CH, _HD), jnp.float32),
            pltpu.VMEM((_HD,), jnp.float32),
            pltpu.VMEM_SHARED((npad, _HD), jnp.float32),
            pltpu.SemaphoreType.DMA,
            pltpu.SemaphoreType.DMA,
        ],
    )
    return f(esrc, edst, aux_tab, c_tab)


def _sc_pass2(esrc, edst, fe_tab, g_tab, c_tab):
    npad = _padn(fe_tab.shape[0])
    rpt = npad // _NSUB
    f = pl.kernel(
        _sc_pass2_body,
        out_type=jax.ShapeDtypeStruct((_NCORE, npad, _HD), jnp.float32),
        mesh=_sc_mesh(),
        scratch_types=[
            pltpu.VMEM((_CH,), jnp.int32),
            pltpu.VMEM((_CH,), jnp.int32),
            pltpu.VMEM((_CH, 2 * _HD), jnp.float32),
            pltpu.VMEM((_CH, _HD), jnp.float32),
            pltpu.VMEM((_CH, _HD), jnp.float32),
            pltpu.VMEM((_HD,), jnp.float32),
            pltpu.VMEM_SHARED((npad, _HD), jnp.float32),
            pltpu.SemaphoreType.DMA,
            pltpu.SemaphoreType.DMA,
        ],
    )
    return f(esrc, edst, fe_tab, g_tab, c_tab)


def _mkproj(a, off):
    h, dh = a.shape
    cols = jnp.arange(_HD)[None, :]
    sel = (cols == (jnp.arange(h * dh) // dh + off)[:, None])
    return a.reshape(-1, 1) * sel.astype(a.dtype)


def _w256(W, al, ar):
    ma = _mkproj(al, 0) + _mkproj(ar, _H)
    return jnp.concatenate([W, W @ ma], axis=1)


def kernel(x, edge_index, W1, al1, ar1, b1, W2, al2, ar2, b2, Wc, bc):
    n = x.shape[0]
    f32 = jnp.float32

    esrc = edge_index[0]
    edst = edge_index[1]
    bigpad = jnp.where(jnp.arange(_HD) < _H, 0.0, _BIG).astype(f32)
    bigpad = bigpad.reshape(1, _HD)

    fe_sds = jax.ShapeDtypeStruct((n, 2 * _HD), f32)
    aux_sds = jax.ShapeDtypeStruct((n, _HD), f32)
    c_sds = jax.ShapeDtypeStruct((1, _HD), f32)

    fe1, aux1, c1 = _tc_call(_prep_body, (fe_sds, aux_sds, c_sds),
                             x, _w256(W1, al1, ar1), bigpad)
    dpart1 = _sc_pass1(esrc, edst, aux1, c1)
    g1 = _tc_call(_rbuild_body, aux_sds, dpart1, aux1)
    apart1 = _sc_pass2(esrc, edst, fe1, g1, c1)

    fe2, aux2, c2 = _tc_call(_prep2_body, (fe_sds, aux_sds, c_sds),
                             apart1, b1.reshape(1, _HD), _w256(W2, al2, ar2),
                             bigpad)
    dpart2 = _sc_pass1(esrc, edst, aux2, c2)
    g2 = _tc_call(_rbuild_body, aux_sds, dpart2, aux2)
    apart2 = _sc_pass2(esrc, edst, fe2, g2, c2)

    return _tc_call(functools.partial(_final_body, n),
                    jax.ShapeDtypeStruct((1, Wc.shape[1]), f32),
                    apart2, b2.reshape(1, _HD), Wc, bc.reshape(1, -1))


# speedup vs baseline: 45.2355x; 1.4962x over previous
import functools

import jax
import jax.numpy as jnp
from jax import lax
from jax.experimental import pallas as pl
from jax.experimental.pallas import tpu as pltpu
from jax.experimental.pallas import tpu_sc as plsc

_H = 8
_DH = 16
_HD = _H * _DH
_NCORE = 2
_NSUB = 16
_CH = 80
_BIG = 1e30


def _prep_body(h_ref, w256_ref, bigpad_ref, fe_ref, aux_ref, c_ref):
    big = jnp.dot(h_ref[...], w256_ref[...], preferred_element_type=jnp.float32)
    fe_ref[...] = big
    aux = big[:, _HD:2 * _HD]
    aux_ref[...] = aux
    cmax = jnp.max(aux, axis=0, keepdims=True)
    csh = jnp.dot(cmax, jnp.eye(_HD, k=-_H, dtype=jnp.float32),
                  preferred_element_type=jnp.float32)
    c_ref[...] = jnp.maximum(cmax + csh, 0.0) + bigpad_ref[...]


def _prep2_body(a_ref, b_ref, w256_ref, bigpad_ref, fe_ref, aux_ref, c_ref):
    n = fe_ref.shape[0]
    h = jnp.maximum(a_ref[0, :n] + a_ref[1, :n] + b_ref[...], 0.0)
    big = jnp.dot(h, w256_ref[...], preferred_element_type=jnp.float32)
    fe_ref[...] = big
    aux = big[:, _HD:2 * _HD]
    aux_ref[...] = aux
    cmax = jnp.max(aux, axis=0, keepdims=True)
    csh = jnp.dot(cmax, jnp.eye(_HD, k=-_H, dtype=jnp.float32),
                  preferred_element_type=jnp.float32)
    c_ref[...] = jnp.maximum(cmax + csh, 0.0) + bigpad_ref[...]


def _rbuild_body(dp_ref, aux_ref, g_ref):
    n = aux_ref.shape[0]
    d = dp_ref[0, :n] + dp_ref[1, :n]
    f32 = jnp.float32
    er_sh = jnp.dot(aux_ref[...], jnp.eye(_HD, k=-_H, dtype=f32),
                    preferred_element_type=f32)
    dsh = jnp.dot(d, jnp.eye(_HD, k=_H, dtype=f32),
                  preferred_element_type=f32)
    lane = lax.broadcasted_iota(jnp.int32, d.shape, 1)
    g_ref[...] = er_sh + jnp.where((lane >= _H) & (lane < 2 * _H),
                                   1.0 / dsh, 0.0)


def _final_body(n, ap_ref, b_ref, wc_ref, bc_ref, o_ref):
    hmean = (jnp.sum(ap_ref[0] + ap_ref[1], axis=0, keepdims=True)
             * (1.0 / n) + b_ref[...])
    o_ref[...] = jnp.dot(hmean, wc_ref[...],
                         preferred_element_type=jnp.float32) + bc_ref[...]


def _tc_call(body, out_shapes, *args):
    return pl.pallas_call(body, out_shape=out_shapes)(*args)


def _vgather(vec, idx):
    dn = lax.GatherDimensionNumbers(offset_dims=(), collapsed_slice_dims=(0,),
                                    start_index_map=(0,))
    return lax.gather(vec, idx.reshape(16, 1), dn, (1,),
                      mode=lax.GatherScatterMode.PROMISE_IN_BOUNDS)


def _idx8():
    return (lax.iota(jnp.int32, 16) & 7) + _H


def _zero_rows(buf, nrows, ncols):
    def zb(j, carry):
        for col in range(0, ncols, 16):
            buf[j, pl.ds(col, 16)] = jnp.zeros((16,), jnp.float32)
        return carry
    lax.fori_loop(0, nrows, zb, 0)


def _sc_pass1_body(esrc, edst, aux_tab, c_tab, dout,
                   src_i, dst_i, s_rows, d_rows, w_rows, c_buf, dsh,
                   sem1, sem2):
    e = esrc.shape[0]
    npad = dsh.shape[0]
    ept = e // (_NCORE * _NSUB)
    nchunk = ept // _CH
    rpt = npad // _NSUB
    ci = lax.axis_index("c")
    si = lax.axis_index("s")

    _zero_rows(w_rows, _CH, _HD)
    for k in range(rpt // _CH):
        pltpu.sync_copy(w_rows, dsh.at[pl.ds(si * rpt + k * _CH, _CH)])
    plsc.subcore_barrier()

    pltpu.sync_copy(c_tab.at[0], c_buf)
    cvec = c_buf[pl.ds(0, 16)]
    idx8 = _idx8()

    def chunk(i, carry):
        base = ci * (e // _NCORE) + si * ept + i * _CH
        pltpu.sync_copy(esrc.at[pl.ds(base, _CH)], src_i)
        pltpu.sync_copy(edst.at[pl.ds(base, _CH)], dst_i)
        cp1 = pltpu.async_copy(aux_tab.at[src_i], s_rows, sem1)
        cp2 = pltpu.async_copy(aux_tab.at[dst_i], d_rows, sem2)
        cp1.wait()
        cp2.wait()

        @plsc.parallel_loop(0, _CH, unroll=4)
        def edge(j):
            s16 = s_rows[j, pl.ds(0, 16)]
            d16 = d_rows[j, pl.ds(0, 16)]
            z = s16 + _vgather(d16, idx8)
            z = jnp.where(z > 0, z, 0.2 * z)
            w_rows[j, pl.ds(0, 16)] = jnp.exp(z - cvec)
        pltpu.sync_copy(w_rows, dsh.at[dst_i], add=True)
        return carry
    lax.fori_loop(0, nchunk, chunk, 0)
    plsc.subcore_barrier()

    for k in range(rpt // _CH):
        pltpu.sync_copy(dsh.at[pl.ds(si * rpt + k * _CH, _CH)], s_rows)
        pltpu.sync_copy(s_rows, dout.at[ci, pl.ds(si * rpt + k * _CH, _CH)])


def _sc_pass2_body(esrc, edst, fe_tab, g_tab, c_tab, aout,
                   src_i, dst_i, fe_rows, g_rows, m_rows, c_buf,
                   ash, sem1, sem2):
    e = esrc.shape[0]
    npad = ash.shape[0]
    ept = e // (_NCORE * _NSUB)
    nchunk = ept // _CH
    rpt = npad // _NSUB
    ci = lax.axis_index("c")
    si = lax.axis_index("s")

    _zero_rows(m_rows, _CH, _HD)
    for k in range(rpt // _CH):
        pltpu.sync_copy(m_rows, ash.at[pl.ds(si * rpt + k * _CH, _CH)])
    plsc.subcore_barrier()

    pltpu.sync_copy(c_tab.at[0], c_buf)
    cvec = c_buf[pl.ds(0, 16)]
    idx8 = _idx8()

    def chunk(i, carry):
        base = ci * (e // _NCORE) + si * ept + i * _CH
        pltpu.sync_copy(esrc.at[pl.ds(base, _CH)], src_i)
        pltpu.sync_copy(edst.at[pl.ds(base, _CH)], dst_i)
        cp1 = pltpu.async_copy(fe_tab.at[src_i], fe_rows, sem1)
        cp2 = pltpu.async_copy(g_tab.at[dst_i], g_rows, sem2)
        cp1.wait()
        cp2.wait()

        @plsc.parallel_loop(0, _CH, unroll=2)
        def edge(j):
            s16 = fe_rows[j, pl.ds(_HD, 16)]
            g16 = g_rows[j, pl.ds(0, 16)]
            z = s16 + g16
            z = jnp.where(z > 0, z, 0.2 * z)
            w = jnp.exp(z - cvec)
            rd = _vgather(g16, idx8)
            coef = w * rd
            for hh in range(_H):
                b = _vgather(coef, jnp.full((16,), hh, jnp.int32))
                m_rows[j, pl.ds(hh * _DH, _DH)] = (
                    fe_rows[j, pl.ds(hh * _DH, _DH)] * b)
        pltpu.sync_copy(m_rows, ash.at[dst_i], add=True)
        return carry
    lax.fori_loop(0, nchunk, chunk, 0)
    plsc.subcore_barrier()

    for k in range(rpt // _CH):
        pltpu.sync_copy(ash.at[pl.ds(si * rpt + k * _CH, _CH)], m_rows)
        pltpu.sync_copy(m_rows, aout.at[ci, pl.ds(si * rpt + k * _CH, _CH)])


def _sc_mesh():
    return plsc.VectorSubcoreMesh(core_axis_name="c", subcore_axis_name="s",
                                  num_cores=_NCORE, num_subcores=_NSUB)


def _padn(n):
    g = _NSUB * _CH
    return ((n + g - 1) // g) * g


def _sc_pass1(esrc, edst, aux_tab, c_tab):
    npad = _padn(aux_tab.shape[0])
    rpt = npad // _NSUB
    f = pl.kernel(
        _sc_pass1_body,
        out_type=jax.ShapeDtypeStruct((_NCORE, npad, _HD), jnp.float32),
        mesh=_sc_mesh(),
        scratch_types=[
            pltpu.VMEM((_CH,), jnp.int32),
            pltpu.VMEM((_CH,), jnp.int32),
            pltpu.VMEM((_CH, _HD), jnp.float32),
            pltpu.VMEM((_CH, _HD), jnp.float32),
            pltpu.VMEM((_CH, _HD), jnp.float32),
            pltpu.VMEM((_HD,), jnp.float32),
            pltpu.VMEM_SHARED((npad, _HD), jnp.float32),
            pltpu.SemaphoreType.DMA,
            pltpu.SemaphoreType.DMA,
        ],
    )
    return f(esrc, edst, aux_tab, c_tab)


def _sc_pass2(esrc, edst, fe_tab, g_tab, c_tab):
    npad = _padn(fe_tab.shape[0])
    rpt = npad // _NSUB
    f = pl.kernel(
        _sc_pass2_body,
        out_type=jax.ShapeDtypeStruct((_NCORE, npad, _HD), jnp.float32),
        mesh=_sc_mesh(),
        scratch_types=[
            pltpu.VMEM((_CH,), jnp.int32),
            pltpu.VMEM((_CH,), jnp.int32),
            pltpu.VMEM((_CH, 2 * _HD), jnp.float32),
            pltpu.VMEM((_CH, _HD), jnp.float32),
            pltpu.VMEM((_CH, _HD), jnp.float32),
            pltpu.VMEM((_HD,), jnp.float32),
            pltpu.VMEM_SHARED((npad, _HD), jnp.float32),
            pltpu.SemaphoreType.DMA,
            pltpu.SemaphoreType.DMA,
        ],
    )
    return f(esrc, edst, fe_tab, g_tab, c_tab)


def _mkproj(a, off):
    h, dh = a.shape
    cols = jnp.arange(_HD)[None, :]
    sel = (cols == (jnp.arange(h * dh) // dh + off)[:, None])
    return a.reshape(-1, 1) * sel.astype(a.dtype)


def _w256(W, al, ar):
    ma = _mkproj(al, 0) + _mkproj(ar, _H)
    return jnp.concatenate([W, W @ ma], axis=1)


def kernel(x, edge_index, W1, al1, ar1, b1, W2, al2, ar2, b2, Wc, bc):
    n = x.shape[0]
    f32 = jnp.float32

    esrc = edge_index[0]
    edst = edge_index[1]
    bigpad = jnp.where(jnp.arange(_HD) < _H, 0.0, _BIG).astype(f32)
    bigpad = bigpad.reshape(1, _HD)

    fe_sds = jax.ShapeDtypeStruct((n, 2 * _HD), f32)
    aux_sds = jax.ShapeDtypeStruct((n, _HD), f32)
    c_sds = jax.ShapeDtypeStruct((1, _HD), f32)

    fe1, aux1, c1 = _tc_call(_prep_body, (fe_sds, aux_sds, c_sds),
                             x, _w256(W1, al1, ar1), bigpad)
    dpart1 = _sc_pass1(esrc, edst, aux1, c1)
    g1 = _tc_call(_rbuild_body, aux_sds, dpart1, aux1)
    apart1 = _sc_pass2(esrc, edst, fe1, g1, c1)

    fe2, aux2, c2 = _tc_call(_prep2_body, (fe_sds, aux_sds, c_sds),
                             apart1, b1.reshape(1, _HD), _w256(W2, al2, ar2),
                             bigpad)
    dpart2 = _sc_pass1(esrc, edst, aux2, c2)
    g2 = _tc_call(_rbuild_body, aux_sds, dpart2, aux2)
    apart2 = _sc_pass2(esrc, edst, fe2, g2, c2)

    return _tc_call(functools.partial(_final_body, n),
                    jax.ShapeDtypeStruct((1, Wc.shape[1]), f32),
                    apart2, b2.reshape(1, _HD), Wc, bc.reshape(1, -1))


# speedup vs baseline: 54.4850x; 1.2045x over previous
import functools

import jax
import jax.numpy as jnp
from jax import lax
from jax.experimental import pallas as pl
from jax.experimental.pallas import tpu as pltpu
from jax.experimental.pallas import tpu_sc as plsc

_H = 8
_DH = 16
_HD = _H * _DH
_NCORE = 2
_NSUB = 16
_CH = 40
_BIG = 1e30


def _prep_body(h_ref, w256_ref, bigpad_ref, fe_ref, aux_ref, c_ref):
    big = jnp.dot(h_ref[...], w256_ref[...], preferred_element_type=jnp.float32)
    fe_ref[...] = big
    aux = big[:, _HD:2 * _HD]
    aux_ref[...] = aux
    cmax = jnp.max(aux, axis=0, keepdims=True)
    csh = jnp.dot(cmax, jnp.eye(_HD, k=-_H, dtype=jnp.float32),
                  preferred_element_type=jnp.float32)
    c_ref[...] = jnp.maximum(cmax + csh, 0.0) + bigpad_ref[...]


def _prep2_body(a_ref, b_ref, w256_ref, bigpad_ref, fe_ref, aux_ref, c_ref):
    n = fe_ref.shape[0]
    h = jnp.maximum(a_ref[0, :n] + a_ref[1, :n] + b_ref[...], 0.0)
    big = jnp.dot(h, w256_ref[...], preferred_element_type=jnp.float32)
    fe_ref[...] = big
    aux = big[:, _HD:2 * _HD]
    aux_ref[...] = aux
    cmax = jnp.max(aux, axis=0, keepdims=True)
    csh = jnp.dot(cmax, jnp.eye(_HD, k=-_H, dtype=jnp.float32),
                  preferred_element_type=jnp.float32)
    c_ref[...] = jnp.maximum(cmax + csh, 0.0) + bigpad_ref[...]


def _rbuild_body(dp_ref, aux_ref, g_ref):
    n = aux_ref.shape[0]
    d = dp_ref[0, :n] + dp_ref[1, :n]
    f32 = jnp.float32
    er_sh = jnp.dot(aux_ref[...], jnp.eye(_HD, k=-_H, dtype=f32),
                    preferred_element_type=f32)
    dsh = jnp.dot(d, jnp.eye(_HD, k=_H, dtype=f32),
                  preferred_element_type=f32)
    lane = lax.broadcasted_iota(jnp.int32, d.shape, 1)
    g_ref[...] = er_sh + jnp.where((lane >= _H) & (lane < 2 * _H),
                                   1.0 / dsh, 0.0)


def _final_body(n, ap_ref, b_ref, wc_ref, bc_ref, o_ref):
    hmean = (jnp.sum(ap_ref[0] + ap_ref[1], axis=0, keepdims=True)
             * (1.0 / n) + b_ref[...])
    o_ref[...] = jnp.dot(hmean, wc_ref[...],
                         preferred_element_type=jnp.float32) + bc_ref[...]


def _tc_call(body, out_shapes, *args):
    return pl.pallas_call(body, out_shape=out_shapes)(*args)


def _vgather(vec, idx):
    dn = lax.GatherDimensionNumbers(offset_dims=(), collapsed_slice_dims=(0,),
                                    start_index_map=(0,))
    return lax.gather(vec, idx.reshape(16, 1), dn, (1,),
                      mode=lax.GatherScatterMode.PROMISE_IN_BOUNDS)


def _idx8():
    return (lax.iota(jnp.int32, 16) & 7) + _H


def _zero_rows(buf, nrows, ncols):
    def zb(j, carry):
        for col in range(0, ncols, 16):
            buf[j, pl.ds(col, 16)] = jnp.zeros((16,), jnp.float32)
        return carry
    lax.fori_loop(0, nrows, zb, 0)


def _sc_pass1_body(esrc, edst, aux_tab, c_tab, dout,
                   src_a, dst_a, src_b, dst_b, s_a, d_a, s_b, d_b,
                   w_rows, c_buf, dsh, g1a, g2a, g1b, g2b):
    e = esrc.shape[0]
    npad = dsh.shape[0]
    ept = e // (_NCORE * _NSUB)
    nchunk = ept // _CH
    npair = nchunk // 2
    rpt = npad // _NSUB
    ci = lax.axis_index("c")
    si = lax.axis_index("s")
    ebase = ci * (e // _NCORE) + si * ept

    _zero_rows(w_rows, _CH, _HD)
    for k in range(rpt // _CH):
        pltpu.sync_copy(w_rows, dsh.at[pl.ds(si * rpt + k * _CH, _CH)])
    plsc.subcore_barrier()

    pltpu.sync_copy(c_tab.at[0], c_buf)
    cvec = c_buf[pl.ds(0, 16)]
    idx8 = _idx8()

    def fire(ch, srcb, dstb, sb, db, s1, s2):
        base = ebase + ch * _CH
        pltpu.sync_copy(esrc.at[pl.ds(base, _CH)], srcb)
        pltpu.sync_copy(edst.at[pl.ds(base, _CH)], dstb)
        pltpu.async_copy(aux_tab.at[srcb], sb, s1)
        pltpu.async_copy(aux_tab.at[dstb], db, s2)

    def process(srcb, dstb, sb, db, s1, s2):
        pltpu.make_async_copy(aux_tab.at[srcb], sb, s1).wait()
        pltpu.make_async_copy(aux_tab.at[dstb], db, s2).wait()

        @plsc.parallel_loop(0, _CH, unroll=4)
        def edge(j):
            s16 = sb[j, pl.ds(0, 16)]
            d16 = db[j, pl.ds(0, 16)]
            z = s16 + _vgather(d16, idx8)
            z = jnp.where(z > 0, z, 0.2 * z)
            w_rows[j, pl.ds(0, 16)] = jnp.exp(z - cvec)
        pltpu.sync_copy(w_rows, dsh.at[dstb], add=True)

    fire(0, src_a, dst_a, s_a, d_a, g1a, g2a)

    def pair(i, carry):
        fire(2 * i + 1, src_b, dst_b, s_b, d_b, g1b, g2b)
        process(src_a, dst_a, s_a, d_a, g1a, g2a)

        @pl.when(i < npair - 1)
        def _():
            fire(2 * i + 2, src_a, dst_a, s_a, d_a, g1a, g2a)
        process(src_b, dst_b, s_b, d_b, g1b, g2b)
        return carry
    lax.fori_loop(0, npair, pair, 0)
    plsc.subcore_barrier()

    for k in range(rpt // _CH):
        pltpu.sync_copy(dsh.at[pl.ds(si * rpt + k * _CH, _CH)], s_a)
        pltpu.sync_copy(s_a, dout.at[ci, pl.ds(si * rpt + k * _CH, _CH)])


def _sc_pass2_body(esrc, edst, fe_tab, g_tab, c_tab, aout,
                   src_a, dst_a, src_b, dst_b, fe_a, g_a, fe_b, g_b,
                   m_rows, c_buf, ash, g1a, g2a, g1b, g2b):
    e = esrc.shape[0]
    npad = ash.shape[0]
    ept = e // (_NCORE * _NSUB)
    nchunk = ept // _CH
    npair = nchunk // 2
    rpt = npad // _NSUB
    ci = lax.axis_index("c")
    si = lax.axis_index("s")
    ebase = ci * (e // _NCORE) + si * ept

    _zero_rows(m_rows, _CH, _HD)
    for k in range(rpt // _CH):
        pltpu.sync_copy(m_rows, ash.at[pl.ds(si * rpt + k * _CH, _CH)])
    plsc.subcore_barrier()

    pltpu.sync_copy(c_tab.at[0], c_buf)
    cvec = c_buf[pl.ds(0, 16)]
    idx8 = _idx8()

    def fire(ch, srcb, dstb, feb, gb, s1, s2):
        base = ebase + ch * _CH
        pltpu.sync_copy(esrc.at[pl.ds(base, _CH)], srcb)
        pltpu.sync_copy(edst.at[pl.ds(base, _CH)], dstb)
        pltpu.async_copy(fe_tab.at[srcb], feb, s1)
        pltpu.async_copy(g_tab.at[dstb], gb, s2)

    def process(srcb, dstb, feb, gb, s1, s2):
        pltpu.make_async_copy(fe_tab.at[srcb], feb, s1).wait()
        pltpu.make_async_copy(g_tab.at[dstb], gb, s2).wait()

        @plsc.parallel_loop(0, _CH, unroll=2)
        def edge(j):
            s16 = feb[j, pl.ds(_HD, 16)]
            g16 = gb[j, pl.ds(0, 16)]
            z = s16 + g16
            z = jnp.where(z > 0, z, 0.2 * z)
            w = jnp.exp(z - cvec)
            rd = _vgather(g16, idx8)
            coef = w * rd
            for hh in range(_H):
                b = _vgather(coef, jnp.full((16,), hh, jnp.int32))
                m_rows[j, pl.ds(hh * _DH, _DH)] = (
                    feb[j, pl.ds(hh * _DH, _DH)] * b)
        pltpu.sync_copy(m_rows, ash.at[dstb], add=True)

    fire(0, src_a, dst_a, fe_a, g_a, g1a, g2a)

    def pair(i, carry):
        fire(2 * i + 1, src_b, dst_b, fe_b, g_b, g1b, g2b)
        process(src_a, dst_a, fe_a, g_a, g1a, g2a)

        @pl.when(i < npair - 1)
        def _():
            fire(2 * i + 2, src_a, dst_a, fe_a, g_a, g1a, g2a)
        process(src_b, dst_b, fe_b, g_b, g1b, g2b)
        return carry
    lax.fori_loop(0, npair, pair, 0)
    plsc.subcore_barrier()

    for k in range(rpt // _CH):
        pltpu.sync_copy(ash.at[pl.ds(si * rpt + k * _CH, _CH)], m_rows)
        pltpu.sync_copy(m_rows, aout.at[ci, pl.ds(si * rpt + k * _CH, _CH)])


def _sc_mesh():
    return plsc.VectorSubcoreMesh(core_axis_name="c", subcore_axis_name="s",
                                  num_cores=_NCORE, num_subcores=_NSUB)


def _padn(n):
    g = _NSUB * _CH
    return ((n + g - 1) // g) * g


def _sc_pass1(esrc, edst, aux_tab, c_tab):
    npad = _padn(aux_tab.shape[0])
    f = pl.kernel(
        _sc_pass1_body,
        out_type=jax.ShapeDtypeStruct((_NCORE, npad, _HD), jnp.float32),
        mesh=_sc_mesh(),
        scratch_types=[
            pltpu.VMEM((_CH,), jnp.int32),
            pltpu.VMEM((_CH,), jnp.int32),
            pltpu.VMEM((_CH,), jnp.int32),
            pltpu.VMEM((_CH,), jnp.int32),
            pltpu.VMEM((_CH, _HD), jnp.float32),
            pltpu.VMEM((_CH, _HD), jnp.float32),
            pltpu.VMEM((_CH, _HD), jnp.float32),
            pltpu.VMEM((_CH, _HD), jnp.float32),
            pltpu.VMEM((_CH, _HD), jnp.float32),
            pltpu.VMEM((_HD,), jnp.float32),
            pltpu.VMEM_SHARED((npad, _HD), jnp.float32),
            pltpu.SemaphoreType.DMA,
            pltpu.SemaphoreType.DMA,
            pltpu.SemaphoreType.DMA,
            pltpu.SemaphoreType.DMA,
        ],
    )
    return f(esrc, edst, aux_tab, c_tab)


def _sc_pass2(esrc, edst, fe_tab, g_tab, c_tab):
    npad = _padn(fe_tab.shape[0])
    f = pl.kernel(
        _sc_pass2_body,
        out_type=jax.ShapeDtypeStruct((_NCORE, npad, _HD), jnp.float32),
        mesh=_sc_mesh(),
        scratch_types=[
            pltpu.VMEM((_CH,), jnp.int32),
            pltpu.VMEM((_CH,), jnp.int32),
            pltpu.VMEM((_CH,), jnp.int32),
            pltpu.VMEM((_CH,), jnp.int32),
            pltpu.VMEM((_CH, 2 * _HD), jnp.float32),
            pltpu.VMEM((_CH, _HD), jnp.float32),
            pltpu.VMEM((_CH, 2 * _HD), jnp.float32),
            pltpu.VMEM((_CH, _HD), jnp.float32),
            pltpu.VMEM((_CH, _HD), jnp.float32),
            pltpu.VMEM((_HD,), jnp.float32),
            pltpu.VMEM_SHARED((npad, _HD), jnp.float32),
            pltpu.SemaphoreType.DMA,
            pltpu.SemaphoreType.DMA,
            pltpu.SemaphoreType.DMA,
            pltpu.SemaphoreType.DMA,
        ],
    )
    return f(esrc, edst, fe_tab, g_tab, c_tab)


def _mkproj(a, off):
    h, dh = a.shape
    cols = jnp.arange(_HD)[None, :]
    sel = (cols == (jnp.arange(h * dh) // dh + off)[:, None])
    return a.reshape(-1, 1) * sel.astype(a.dtype)


def _w256(W, al, ar):
    ma = _mkproj(al, 0) + _mkproj(ar, _H)
    return jnp.concatenate([W, W @ ma], axis=1)


def kernel(x, edge_index, W1, al1, ar1, b1, W2, al2, ar2, b2, Wc, bc):
    n = x.shape[0]
    f32 = jnp.float32

    esrc = edge_index[0]
    edst = edge_index[1]
    bigpad = jnp.where(jnp.arange(_HD) < _H, 0.0, _BIG).astype(f32)
    bigpad = bigpad.reshape(1, _HD)

    fe_sds = jax.ShapeDtypeStruct((n, 2 * _HD), f32)
    aux_sds = jax.ShapeDtypeStruct((n, _HD), f32)
    c_sds = jax.ShapeDtypeStruct((1, _HD), f32)

    fe1, aux1, c1 = _tc_call(_prep_body, (fe_sds, aux_sds, c_sds),
                             x, _w256(W1, al1, ar1), bigpad)
    dpart1 = _sc_pass1(esrc, edst, aux1, c1)
    g1 = _tc_call(_rbuild_body, aux_sds, dpart1, aux1)
    apart1 = _sc_pass2(esrc, edst, fe1, g1, c1)

    fe2, aux2, c2 = _tc_call(_prep2_body, (fe_sds, aux_sds, c_sds),
                             apart1, b1.reshape(1, _HD), _w256(W2, al2, ar2),
                             bigpad)
    dpart2 = _sc_pass1(esrc, edst, aux2, c2)
    g2 = _tc_call(_rbuild_body, aux_sds, dpart2, aux2)
    apart2 = _sc_pass2(esrc, edst, fe2, g2, c2)

    return _tc_call(functools.partial(_final_body, n),
                    jax.ShapeDtypeStruct((1, Wc.shape[1]), f32),
                    apart2, b2.reshape(1, _HD), Wc, bc.reshape(1, -1))


# speedup vs baseline: 54.5954x; 1.0020x over previous
import functools

import jax
import jax.numpy as jnp
from jax import lax
from jax.experimental import pallas as pl
from jax.experimental.pallas import tpu as pltpu
from jax.experimental.pallas import tpu_sc as plsc

_H = 8
_DH = 16
_HD = _H * _DH
_NCORE = 2
_NSUB = 16
_CH = 40
_BIG = 1e30


def _prep_body(h_ref, w256_ref, bigpad_ref, fe_ref, aux_ref, c_ref):
    big = jnp.dot(h_ref[...], w256_ref[...], preferred_element_type=jnp.float32)
    fe_ref[...] = big
    aux = big[:, _HD:2 * _HD]
    aux_ref[...] = aux
    cmax = jnp.max(aux, axis=0, keepdims=True)
    csh = jnp.dot(cmax, jnp.eye(_HD, k=-_H, dtype=jnp.float32),
                  preferred_element_type=jnp.float32)
    c_ref[...] = jnp.maximum(cmax + csh, 0.0) + bigpad_ref[...]


def _prep2_body(a_ref, b_ref, w256_ref, bigpad_ref, fe_ref, aux_ref, c_ref):
    n = fe_ref.shape[0]
    h = jnp.maximum(a_ref[0, :n] + a_ref[1, :n] + b_ref[...], 0.0)
    big = jnp.dot(h, w256_ref[...], preferred_element_type=jnp.float32)
    fe_ref[...] = big
    aux = big[:, _HD:2 * _HD]
    aux_ref[...] = aux
    cmax = jnp.max(aux, axis=0, keepdims=True)
    csh = jnp.dot(cmax, jnp.eye(_HD, k=-_H, dtype=jnp.float32),
                  preferred_element_type=jnp.float32)
    c_ref[...] = jnp.maximum(cmax + csh, 0.0) + bigpad_ref[...]


def _rbuild_body(dp_ref, aux_ref, g_ref):
    n = aux_ref.shape[0]
    d = dp_ref[0, :n] + dp_ref[1, :n]
    f32 = jnp.float32
    er_sh = jnp.dot(aux_ref[...], jnp.eye(_HD, k=-_H, dtype=f32),
                    preferred_element_type=f32)
    dsh = jnp.dot(d, jnp.eye(_HD, k=_H, dtype=f32),
                  preferred_element_type=f32)
    lane = lax.broadcasted_iota(jnp.int32, d.shape, 1)
    g_ref[...] = er_sh + jnp.where((lane >= _H) & (lane < 2 * _H),
                                   1.0 / dsh, 0.0)


def _final_body(n, ap_ref, b_ref, wc_ref, bc_ref, o_ref):
    hmean = (jnp.sum(ap_ref[0] + ap_ref[1], axis=0, keepdims=True)
             * (1.0 / n) + b_ref[...])
    o_ref[...] = jnp.dot(hmean, wc_ref[...],
                         preferred_element_type=jnp.float32) + bc_ref[...]


def _tc_call(body, out_shapes, *args):
    return pl.pallas_call(body, out_shape=out_shapes)(*args)


def _vgather(vec, idx):
    dn = lax.GatherDimensionNumbers(offset_dims=(), collapsed_slice_dims=(0,),
                                    start_index_map=(0,))
    return lax.gather(vec, idx.reshape(16, 1), dn, (1,),
                      mode=lax.GatherScatterMode.PROMISE_IN_BOUNDS)


def _idx8():
    return (lax.iota(jnp.int32, 16) & 7) + _H


def _zero_rows(buf, nrows, ncols):
    def zb(j, carry):
        for col in range(0, ncols, 16):
            buf[j, pl.ds(col, 16)] = jnp.zeros((16,), jnp.float32)
        return carry
    lax.fori_loop(0, nrows, zb, 0)


def _sc_pass1_body(esrc, edst, aux_tab, c_tab, dout,
                   src_a, dst_a, src_b, dst_b, s_a, d_a, s_b, d_b,
                   w_rows, c_buf, dsh, g1a, g2a, g1b, g2b):
    e = esrc.shape[0]
    npad = dsh.shape[0]
    ept = e // (_NCORE * _NSUB)
    nchunk = ept // _CH
    npair = nchunk // 2
    rpt = npad // _NSUB
    ci = lax.axis_index("c")
    si = lax.axis_index("s")
    ebase = ci * (e // _NCORE) + si * ept

    _zero_rows(w_rows, _CH, _HD)
    for k in range(rpt // _CH):
        pltpu.sync_copy(w_rows, dsh.at[pl.ds(si * rpt + k * _CH, _CH)])
    plsc.subcore_barrier()

    pltpu.sync_copy(c_tab.at[0], c_buf)
    cvec = c_buf[pl.ds(0, 16)]
    idx8 = _idx8()

    def fire(ch, srcb, dstb, sb, db, s1, s2):
        base = ebase + ch * _CH
        pltpu.sync_copy(esrc.at[pl.ds(base, _CH)], srcb)
        pltpu.sync_copy(edst.at[pl.ds(base, _CH)], dstb)
        pltpu.async_copy(aux_tab.at[srcb], sb, s1)
        pltpu.async_copy(aux_tab.at[dstb], db, s2)

    def process(srcb, dstb, sb, db, s1, s2):
        pltpu.make_async_copy(aux_tab.at[srcb], sb, s1).wait()
        pltpu.make_async_copy(aux_tab.at[dstb], db, s2).wait()

        @plsc.parallel_loop(0, _CH, unroll=4)
        def edge(j):
            s16 = sb[j, pl.ds(0, 16)]
            d16 = db[j, pl.ds(0, 16)]
            z = s16 + _vgather(d16, idx8)
            z = jnp.where(z > 0, z, 0.2 * z)
            w_rows[j, pl.ds(0, 16)] = jnp.exp(z - cvec)
        pltpu.sync_copy(w_rows, dsh.at[dstb], add=True)

    fire(0, src_a, dst_a, s_a, d_a, g1a, g2a)

    def pair(i, carry):
        fire(2 * i + 1, src_b, dst_b, s_b, d_b, g1b, g2b)
        process(src_a, dst_a, s_a, d_a, g1a, g2a)

        @pl.when(i < npair - 1)
        def _():
            fire(2 * i + 2, src_a, dst_a, s_a, d_a, g1a, g2a)
        process(src_b, dst_b, s_b, d_b, g1b, g2b)
        return carry
    lax.fori_loop(0, npair, pair, 0)
    plsc.subcore_barrier()

    for k in range(rpt // _CH):
        pltpu.sync_copy(dsh.at[pl.ds(si * rpt + k * _CH, _CH)], s_a)
        pltpu.sync_copy(s_a, dout.at[ci, pl.ds(si * rpt + k * _CH, _CH)])


def _sc_pass2_body(esrc, edst, fe_tab, g_tab, c_tab, aout,
                   src_a, dst_a, src_b, dst_b, fe_a, g_a, fe_b, g_b,
                   m_rows, c_buf, ash, g1a, g2a, g1b, g2b):
    e = esrc.shape[0]
    npad = ash.shape[0]
    ept = e // (_NCORE * _NSUB)
    nchunk = ept // _CH
    npair = nchunk // 2
    rpt = npad // _NSUB
    ci = lax.axis_index("c")
    si = lax.axis_index("s")
    ebase = ci * (e // _NCORE) + si * ept

    _zero_rows(m_rows, _CH, _HD)
    for k in range(rpt // _CH):
        pltpu.sync_copy(m_rows, ash.at[pl.ds(si * rpt + k * _CH, _CH)])
    plsc.subcore_barrier()

    pltpu.sync_copy(c_tab.at[0], c_buf)
    cvec = c_buf[pl.ds(0, 16)]
    idx8 = _idx8()

    def fire(ch, srcb, dstb, feb, gb, s1, s2):
        base = ebase + ch * _CH
        pltpu.sync_copy(esrc.at[pl.ds(base, _CH)], srcb)
        pltpu.sync_copy(edst.at[pl.ds(base, _CH)], dstb)
        pltpu.async_copy(fe_tab.at[srcb], feb, s1)
        pltpu.async_copy(g_tab.at[dstb], gb, s2)

    def process(srcb, dstb, feb, gb, s1, s2):
        pltpu.make_async_copy(fe_tab.at[srcb], feb, s1).wait()
        pltpu.make_async_copy(g_tab.at[dstb], gb, s2).wait()

        @plsc.parallel_loop(0, _CH, unroll=4)
        def edge(j):
            s16 = feb[j, pl.ds(_HD, 16)]
            g16 = gb[j, pl.ds(0, 16)]
            z = s16 + g16
            z = jnp.where(z > 0, z, 0.2 * z)
            w = jnp.exp(z - cvec)
            rd = _vgather(g16, idx8)
            coef = w * rd
            for hh in range(_H):
                b = _vgather(coef, jnp.full((16,), hh, jnp.int32))
                m_rows[j, pl.ds(hh * _DH, _DH)] = (
                    feb[j, pl.ds(hh * _DH, _DH)] * b)
        pltpu.sync_copy(m_rows, ash.at[dstb], add=True)

    fire(0, src_a, dst_a, fe_a, g_a, g1a, g2a)

    def pair(i, carry):
        fire(2 * i + 1, src_b, dst_b, fe_b, g_b, g1b, g2b)
        process(src_a, dst_a, fe_a, g_a, g1a, g2a)

        @pl.when(i < npair - 1)
        def _():
            fire(2 * i + 2, src_a, dst_a, fe_a, g_a, g1a, g2a)
        process(src_b, dst_b, fe_b, g_b, g1b, g2b)
        return carry
    lax.fori_loop(0, npair, pair, 0)
    plsc.subcore_barrier()

    for k in range(rpt // _CH):
        pltpu.sync_copy(ash.at[pl.ds(si * rpt + k * _CH, _CH)], m_rows)
        pltpu.sync_copy(m_rows, aout.at[ci, pl.ds(si * rpt + k * _CH, _CH)])


def _sc_mesh():
    return plsc.VectorSubcoreMesh(core_axis_name="c", subcore_axis_name="s",
                                  num_cores=_NCORE, num_subcores=_NSUB)


def _padn(n):
    g = _NSUB * _CH
    return ((n + g - 1) // g) * g


def _sc_pass1(esrc, edst, aux_tab, c_tab):
    npad = _padn(aux_tab.shape[0])
    f = pl.kernel(
        _sc_pass1_body,
        out_type=jax.ShapeDtypeStruct((_NCORE, npad, _HD), jnp.float32),
        mesh=_sc_mesh(),
        scratch_types=[
            pltpu.VMEM((_CH,), jnp.int32),
            pltpu.VMEM((_CH,), jnp.int32),
            pltpu.VMEM((_CH,), jnp.int32),
            pltpu.VMEM((_CH,), jnp.int32),
            pltpu.VMEM((_CH, _HD), jnp.float32),
            pltpu.VMEM((_CH, _HD), jnp.float32),
            pltpu.VMEM((_CH, _HD), jnp.float32),
            pltpu.VMEM((_CH, _HD), jnp.float32),
            pltpu.VMEM((_CH, _HD), jnp.float32),
            pltpu.VMEM((_HD,), jnp.float32),
            pltpu.VMEM_SHARED((npad, _HD), jnp.float32),
            pltpu.SemaphoreType.DMA,
            pltpu.SemaphoreType.DMA,
            pltpu.SemaphoreType.DMA,
            pltpu.SemaphoreType.DMA,
        ],
    )
    return f(esrc, edst, aux_tab, c_tab)


def _sc_pass2(esrc, edst, fe_tab, g_tab, c_tab):
    npad = _padn(fe_tab.shape[0])
    f = pl.kernel(
        _sc_pass2_body,
        out_type=jax.ShapeDtypeStruct((_NCORE, npad, _HD), jnp.float32),
        mesh=_sc_mesh(),
        scratch_types=[
            pltpu.VMEM((_CH,), jnp.int32),
            pltpu.VMEM((_CH,), jnp.int32),
            pltpu.VMEM((_CH,), jnp.int32),
            pltpu.VMEM((_CH,), jnp.int32),
            pltpu.VMEM((_CH, 2 * _HD), jnp.float32),
            pltpu.VMEM((_CH, _HD), jnp.float32),
            pltpu.VMEM((_CH, 2 * _HD), jnp.float32),
            pltpu.VMEM((_CH, _HD), jnp.float32),
            pltpu.VMEM((_CH, _HD), jnp.float32),
            pltpu.VMEM((_HD,), jnp.float32),
            pltpu.VMEM_SHARED((npad, _HD), jnp.float32),
            pltpu.SemaphoreType.DMA,
            pltpu.SemaphoreType.DMA,
            pltpu.SemaphoreType.DMA,
            pltpu.SemaphoreType.DMA,
        ],
    )
    return f(esrc, edst, fe_tab, g_tab, c_tab)


def _mkproj(a, off):
    h, dh = a.shape
    cols = jnp.arange(_HD)[None, :]
    sel = (cols == (jnp.arange(h * dh) // dh + off)[:, None])
    return a.reshape(-1, 1) * sel.astype(a.dtype)


def _w256(W, al, ar):
    ma = _mkproj(al, 0) + _mkproj(ar, _H)
    return jnp.concatenate([W, W @ ma], axis=1)


def kernel(x, edge_index, W1, al1, ar1, b1, W2, al2, ar2, b2, Wc, bc):
    n = x.shape[0]
    f32 = jnp.float32

    esrc = edge_index[0]
    edst = edge_index[1]
    bigpad = jnp.where(jnp.arange(_HD) < _H, 0.0, _BIG).astype(f32)
    bigpad = bigpad.reshape(1, _HD)

    fe_sds = jax.ShapeDtypeStruct((n, 2 * _HD), f32)
    aux_sds = jax.ShapeDtypeStruct((n, _HD), f32)
    c_sds = jax.ShapeDtypeStruct((1, _HD), f32)

    fe1, aux1, c1 = _tc_call(_prep_body, (fe_sds, aux_sds, c_sds),
                             x, _w256(W1, al1, ar1), bigpad)
    dpart1 = _sc_pass1(esrc, edst, aux1, c1)
    g1 = _tc_call(_rbuild_body, aux_sds, dpart1, aux1)
    apart1 = _sc_pass2(esrc, edst, fe1, g1, c1)

    fe2, aux2, c2 = _tc_call(_prep2_body, (fe_sds, aux_sds, c_sds),
                             apart1, b1.reshape(1, _HD), _w256(W2, al2, ar2),
                             bigpad)
    dpart2 = _sc_pass1(esrc, edst, aux2, c2)
    g2 = _tc_call(_rbuild_body, aux_sds, dpart2, aux2)
    apart2 = _sc_pass2(esrc, edst, fe2, g2, c2)

    return _tc_call(functools.partial(_final_body, n),
                    jax.ShapeDtypeStruct((1, Wc.shape[1]), f32),
                    apart2, b2.reshape(1, _HD), Wc, bc.reshape(1, -1))


# speedup vs baseline: 75.2409x; 1.3782x over previous
import functools

import jax
import jax.numpy as jnp
from jax import lax
from jax.experimental import pallas as pl
from jax.experimental.pallas import tpu as pltpu
from jax.experimental.pallas import tpu_sc as plsc

_H = 8
_DH = 16
_HD = _H * _DH
_NCORE = 2
_NSUB = 16
_CH = 40
_BIG = 1e30
_CHE = 4000


def _prep_common(big, fe_ref, aux_ref, tcol_ref, c_ref, bigpad):
    fe_ref[...] = big
    aux = big[:, _HD:2 * _HD]
    aux_ref[...] = aux
    e16 = (lax.broadcasted_iota(jnp.int32, (16, _HD), 0)
           == lax.broadcasted_iota(jnp.int32, (16, _HD), 1)).astype(jnp.float32)
    tcol_ref[...] = lax.dot_general(e16, aux, (((1,), (1,)), ((), ())),
                                    preferred_element_type=jnp.float32)
    cmax = jnp.max(aux, axis=0, keepdims=True)
    csh = jnp.dot(cmax, jnp.eye(_HD, k=-_H, dtype=jnp.float32),
                  preferred_element_type=jnp.float32)
    c_ref[...] = jnp.maximum(cmax + csh, 0.0) + bigpad


def _prep_body(h_ref, w256_ref, bigpad_ref, fe_ref, aux_ref, tcol_ref, c_ref):
    big = jnp.dot(h_ref[...], w256_ref[...], preferred_element_type=jnp.float32)
    _prep_common(big, fe_ref, aux_ref, tcol_ref, c_ref, bigpad_ref[...])


def _prep2_body(a_ref, b_ref, w256_ref, bigpad_ref, fe_ref, aux_ref, tcol_ref,
                c_ref):
    n = fe_ref.shape[0]
    h = jnp.maximum(a_ref[0, :n] + a_ref[1, :n] + b_ref[...], 0.0)
    big = jnp.dot(h, w256_ref[...], preferred_element_type=jnp.float32)
    _prep_common(big, fe_ref, aux_ref, tcol_ref, c_ref, bigpad_ref[...])


def _rbuild_body(dp_ref, aux_ref, g_ref):
    n = aux_ref.shape[0]
    f32 = jnp.float32
    s16 = dp_ref[0] + dp_ref[1]
    r8 = lax.broadcasted_iota(jnp.int32, (8, 16), 0)
    c16 = lax.broadcasted_iota(jnp.int32, (8, 16), 1)
    pm = (c16 // 2 == r8).astype(f32)
    dht = lax.dot_general(s16, pm, (((0,), (1,)), ((), ())),
                          preferred_element_type=f32)[:n]
    r2 = lax.broadcasted_iota(jnp.int32, (8, _HD), 0)
    c2 = lax.broadcasted_iota(jnp.int32, (8, _HD), 1)
    e8 = (c2 == r2 + _H).astype(f32)
    dpad = lax.dot_general(dht, e8, (((1,), (0,)), ((), ())),
                           preferred_element_type=f32)
    er_sh = jnp.dot(aux_ref[...], jnp.eye(_HD, k=-_H, dtype=f32),
                    preferred_element_type=f32)
    lane = lax.broadcasted_iota(jnp.int32, (n, _HD), 1)
    g_ref[...] = er_sh + jnp.where((lane >= _H) & (lane < 2 * _H),
                                   1.0 / dpad, 0.0)


def _final_body(n, ap_ref, b_ref, wc_ref, bc_ref, o_ref):
    hmean = (jnp.sum(ap_ref[0] + ap_ref[1], axis=0, keepdims=True)
             * (1.0 / n) + b_ref[...])
    o_ref[...] = jnp.dot(hmean, wc_ref[...],
                         preferred_element_type=jnp.float32) + bc_ref[...]


def _tc_call(body, out_shapes, *args):
    return pl.pallas_call(body, out_shape=out_shapes)(*args)


def _vgather(vec, idx):
    dn = lax.GatherDimensionNumbers(offset_dims=(), collapsed_slice_dims=(0,),
                                    start_index_map=(0,))
    return lax.gather(vec, idx.reshape(16, 1), dn, (1,),
                      mode=lax.GatherScatterMode.PROMISE_IN_BOUNDS)


def _idx8():
    return (lax.iota(jnp.int32, 16) & 7) + _H


def _zero_rows(buf, nrows, ncols):
    def zb(j, carry):
        for col in range(0, ncols, 16):
            buf[j, pl.ds(col, 16)] = jnp.zeros((16,), jnp.float32)
        return carry
    lax.fori_loop(0, nrows, zb, 0)


def _sc_pass1_body(esrc, edst, tflat, c_tab, dflat,
                   src_a, dst_a, src_b, dst_b, el_t, er_t, den_t, c_buf,
                   g1a, g2a, g1b, g2b):
    e = esrc.shape[0]
    n = el_t.shape[0]
    nd = den_t.shape[0]
    ept = e // 4
    nchunk = ept // _CHE
    npair = nchunk // 2
    ci = lax.axis_index("c")
    si = lax.axis_index("s")
    hh = si // 2
    qq = ci * 2 + (si % 2)
    ebase = qq * ept

    def zd(k, carry):
        den_t[pl.ds(k * 16, 16)] = jnp.zeros((16,), jnp.float32)
        return carry
    lax.fori_loop(0, nd // 16, zd, 0)

    pltpu.sync_copy(tflat.at[pl.ds(pl.multiple_of(hh * n, 8), n)], el_t)
    pltpu.sync_copy(tflat.at[pl.ds(pl.multiple_of((_H + hh) * n, 8), n)], er_t)
    pltpu.sync_copy(c_tab.at[0], c_buf)
    ch = _vgather(c_buf[pl.ds(0, 16)], jnp.full((16,), hh, jnp.int32))

    def fire(chk, srcb, dstb, s1, s2):
        base = ebase + chk * _CHE
        pltpu.async_copy(esrc.at[pl.ds(base, _CHE)], srcb, s1)
        pltpu.async_copy(edst.at[pl.ds(base, _CHE)], dstb, s2)

    def process(srcb, dstb, s1, s2):
        pltpu.make_async_copy(esrc.at[pl.ds(0, _CHE)], srcb, s1).wait()
        pltpu.make_async_copy(edst.at[pl.ds(0, _CHE)], dstb, s2).wait()

        def grp(g, carry):
            s16i = srcb[pl.ds(g * 16, 16)]
            d16i = dstb[pl.ds(g * 16, 16)]
            z = (plsc.load_gather(el_t, [s16i])
                 + plsc.load_gather(er_t, [d16i]))
            z = jnp.where(z > 0, z, 0.2 * z)
            plsc.addupdate_scatter(den_t, [d16i], jnp.exp(z - ch))
            return carry
        lax.fori_loop(0, _CHE // 16, grp, 0)

    fire(0, src_a, dst_a, g1a, g2a)

    def pair(i, carry):
        fire(2 * i + 1, src_b, dst_b, g1b, g2b)
        process(src_a, dst_a, g1a, g2a)

        @pl.when(i < npair - 1)
        def _():
            fire(2 * i + 2, src_a, dst_a, g1a, g2a)
        process(src_b, dst_b, g1b, g2b)
        return carry
    lax.fori_loop(0, npair, pair, 0)

    wid = ci * _NSUB + si
    pltpu.sync_copy(den_t, dflat.at[pl.ds(pl.multiple_of(wid * nd, 8), nd)])


def _sc_pass2_body(esrc, edst, fe_tab, g_tab, c_tab, aout,
                   src_a, dst_a, src_b, dst_b, fe_a, g_a, fe_b, g_b,
                   m_rows, c_buf, ash, g1a, g2a, g1b, g2b):
    e = esrc.shape[0]
    npad = ash.shape[0]
    ept = e // (_NCORE * _NSUB)
    nchunk = ept // _CH
    npair = nchunk // 2
    rpt = npad // _NSUB
    ci = lax.axis_index("c")
    si = lax.axis_index("s")
    ebase = ci * (e // _NCORE) + si * ept

    _zero_rows(m_rows, _CH, _HD)
    for k in range(rpt // _CH):
        pltpu.sync_copy(m_rows, ash.at[pl.ds(si * rpt + k * _CH, _CH)])
    plsc.subcore_barrier()

    pltpu.sync_copy(c_tab.at[0], c_buf)
    cvec = c_buf[pl.ds(0, 16)]
    idx8 = _idx8()

    def fire(ch, srcb, dstb, feb, gb, s1, s2):
        base = ebase + ch * _CH
        pltpu.sync_copy(esrc.at[pl.ds(base, _CH)], srcb)
        pltpu.sync_copy(edst.at[pl.ds(base, _CH)], dstb)
        pltpu.async_copy(fe_tab.at[srcb], feb, s1)
        pltpu.async_copy(g_tab.at[dstb], gb, s2)

    def process(srcb, dstb, feb, gb, s1, s2):
        pltpu.make_async_copy(fe_tab.at[srcb], feb, s1).wait()
        pltpu.make_async_copy(g_tab.at[dstb], gb, s2).wait()

        @plsc.parallel_loop(0, _CH, unroll=4)
        def edge(j):
            s16 = feb[j, pl.ds(_HD, 16)]
            g16 = gb[j, pl.ds(0, 16)]
            z = s16 + g16
            z = jnp.where(z > 0, z, 0.2 * z)
            w = jnp.exp(z - cvec)
            rd = _vgather(g16, idx8)
            coef = w * rd
            for hh in range(_H):
                b = _vgather(coef, jnp.full((16,), hh, jnp.int32))
                m_rows[j, pl.ds(hh * _DH, _DH)] = (
                    feb[j, pl.ds(hh * _DH, _DH)] * b)
        pltpu.sync_copy(m_rows, ash.at[dstb], add=True)

    fire(0, src_a, dst_a, fe_a, g_a, g1a, g2a)

    def pair(i, carry):
        fire(2 * i + 1, src_b, dst_b, fe_b, g_b, g1b, g2b)
        process(src_a, dst_a, fe_a, g_a, g1a, g2a)

        @pl.when(i < npair - 1)
        def _():
            fire(2 * i + 2, src_a, dst_a, fe_a, g_a, g1a, g2a)
        process(src_b, dst_b, fe_b, g_b, g1b, g2b)
        return carry
    lax.fori_loop(0, npair, pair, 0)
    plsc.subcore_barrier()

    for k in range(rpt // _CH):
        pltpu.sync_copy(ash.at[pl.ds(si * rpt + k * _CH, _CH)], m_rows)
        pltpu.sync_copy(m_rows, aout.at[ci, pl.ds(si * rpt + k * _CH, _CH)])


def _sc_mesh():
    return plsc.VectorSubcoreMesh(core_axis_name="c", subcore_axis_name="s",
                                  num_cores=_NCORE, num_subcores=_NSUB)


def _padn(n):
    g = _NSUB * _CH
    return ((n + g - 1) // g) * g


def _sc_pass1(esrc, edst, tflat, c_tab, n):
    nd = _padn(n)
    f = pl.kernel(
        _sc_pass1_body,
        out_type=jax.ShapeDtypeStruct((_NCORE * _NSUB * nd,), jnp.float32),
        mesh=_sc_mesh(),
        compiler_params=pltpu.CompilerParams(needs_layout_passes=False),
        scratch_types=[
            pltpu.VMEM((_CHE,), jnp.int32),
            pltpu.VMEM((_CHE,), jnp.int32),
            pltpu.VMEM((_CHE,), jnp.int32),
            pltpu.VMEM((_CHE,), jnp.int32),
            pltpu.VMEM((n,), jnp.float32),
            pltpu.VMEM((n,), jnp.float32),
            pltpu.VMEM((nd,), jnp.float32),
            pltpu.VMEM((_HD,), jnp.float32),
            pltpu.SemaphoreType.DMA,
            pltpu.SemaphoreType.DMA,
            pltpu.SemaphoreType.DMA,
            pltpu.SemaphoreType.DMA,
        ],
    )
    return f(esrc, edst, tflat, c_tab).reshape(_NCORE, _NSUB, nd)


def _sc_pass2(esrc, edst, fe_tab, g_tab, c_tab):
    npad = _padn(fe_tab.shape[0])
    f = pl.kernel(
        _sc_pass2_body,
        out_type=jax.ShapeDtypeStruct((_NCORE, npad, _HD), jnp.float32),
        mesh=_sc_mesh(),
        scratch_types=[
            pltpu.VMEM((_CH,), jnp.int32),
            pltpu.VMEM((_CH,), jnp.int32),
            pltpu.VMEM((_CH,), jnp.int32),
            pltpu.VMEM((_CH,), jnp.int32),
            pltpu.VMEM((_CH, 2 * _HD), jnp.float32),
            pltpu.VMEM((_CH, _HD), jnp.float32),
            pltpu.VMEM((_CH, 2 * _HD), jnp.float32),
            pltpu.VMEM((_CH, _HD), jnp.float32),
            pltpu.VMEM((_CH, _HD), jnp.float32),
            pltpu.VMEM((_HD,), jnp.float32),
            pltpu.VMEM_SHARED((npad, _HD), jnp.float32),
            pltpu.SemaphoreType.DMA,
            pltpu.SemaphoreType.DMA,
            pltpu.SemaphoreType.DMA,
            pltpu.SemaphoreType.DMA,
        ],
    )
    return f(esrc, edst, fe_tab, g_tab, c_tab)


def _mkproj(a, off):
    h, dh = a.shape
    cols = jnp.arange(_HD)[None, :]
    sel = (cols == (jnp.arange(h * dh) // dh + off)[:, None])
    return a.reshape(-1, 1) * sel.astype(a.dtype)


def _w256(W, al, ar):
    ma = _mkproj(al, 0) + _mkproj(ar, _H)
    return jnp.concatenate([W, W @ ma], axis=1)


def kernel(x, edge_index, W1, al1, ar1, b1, W2, al2, ar2, b2, Wc, bc):
    n = x.shape[0]
    f32 = jnp.float32

    esrc = edge_index[0]
    edst = edge_index[1]
    bigpad = jnp.where(jnp.arange(_HD) < _H, 0.0, _BIG).astype(f32)
    bigpad = bigpad.reshape(1, _HD)

    fe_sds = jax.ShapeDtypeStruct((n, 2 * _HD), f32)
    aux_sds = jax.ShapeDtypeStruct((n, _HD), f32)
    tcol_sds = jax.ShapeDtypeStruct((16, n), f32)
    c_sds = jax.ShapeDtypeStruct((1, _HD), f32)

    fe1, aux1, tcol1, c1 = _tc_call(_prep_body,
                                    (fe_sds, aux_sds, tcol_sds, c_sds),
                                    x, _w256(W1, al1, ar1), bigpad)
    dpart1 = _sc_pass1(esrc, edst, tcol1.reshape(-1), c1, n)
    g1 = _tc_call(_rbuild_body, aux_sds, dpart1, aux1)
    apart1 = _sc_pass2(esrc, edst, fe1, g1, c1)

    fe2, aux2, tcol2, c2 = _tc_call(_prep2_body,
                                    (fe_sds, aux_sds, tcol_sds, c_sds),
                                    apart1, b1.reshape(1, _HD),
                                    _w256(W2, al2, ar2), bigpad)
    dpart2 = _sc_pass1(esrc, edst, tcol2.reshape(-1), c2, n)
    g2 = _tc_call(_rbuild_body, aux_sds, dpart2, aux2)
    apart2 = _sc_pass2(esrc, edst, fe2, g2, c2)

    return _tc_call(functools.partial(_final_body, n),
                    jax.ShapeDtypeStruct((1, Wc.shape[1]), f32),
                    apart2, b2.reshape(1, _HD), Wc, bc.reshape(1, -1))
